# baseline (speedup 1.0000x reference)
import functools

import jax
import jax.numpy as jnp
from jax import lax
from jax.experimental import pallas as pl
from jax.experimental.pallas import tpu as pltpu
from jax.experimental.pallas import tpu_sc as plsc

N_P = 10000
P_ROWS = 10240
NE = 320000
NC, NS = 2, 16
NW = NC * NS
CHUNK = 128
NE_PAD = 327680
CPT = NE_PAD // (NW * CHUNK)
TROWS = P_ROWS // NS
CNT_W = 16
F = 64


NBUF = 2
KB = 4
NWIN = CPT // KB


def _sc_msg_body(table, srcm, dstm, zeros, out, src_v, dst_v, rows,
                 g0, g1, s0, s1, acc):
    gsems = (g0, g1)
    ssems = (s0, s1)
    c = lax.axis_index("c")
    s = lax.axis_index("s")
    wid = s * NC + c
    pltpu.sync_copy(zeros, acc.at[pl.ds(s * TROWS, TROWS)])
    pltpu.sync_copy(srcm.at[wid], src_v)
    pltpu.sync_copy(dstm.at[wid], dst_v)
    plsc.subcore_barrier()

    def body(j, carry):
        for b in range(NBUF):
            w = NBUF * j + b

            @pl.when(j > 0)
            def _():
                pltpu.make_async_copy(
                    rows.at[b], acc.at[dst_v.at[w - NBUF]], ssems[b]).wait()

            pltpu.async_copy(table.at[src_v.at[w]], rows.at[b], gsems[b])
        for b in range(NBUF):
            w = NBUF * j + b
            pltpu.make_async_copy(
                table.at[src_v.at[w]], rows.at[b], gsems[b]).wait()
            pltpu.async_copy(rows.at[b], acc.at[dst_v.at[w]], ssems[b],
                             add=True)
        return carry

    lax.fori_loop(0, NWIN // NBUF, body, 0)
    for b in range(NBUF):
        pltpu.make_async_copy(
            rows.at[b], acc.at[dst_v.at[NWIN - NBUF + b]], ssems[b]).wait()
    plsc.subcore_barrier()
    pltpu.sync_copy(acc.at[pl.ds(s * TROWS, TROWS)], out.at[c, s])


def _sc_cnt_body(dstm, zeros, ones, out, dst_v, ones_v, cnt):
    c = lax.axis_index("c")
    s = lax.axis_index("s")
    wid = s * NC + c
    pltpu.sync_copy(zeros, cnt.at[pl.ds(s * 2 * TROWS, 2 * TROWS)])
    pltpu.sync_copy(dstm.at[wid], dst_v)
    pltpu.sync_copy(ones, ones_v)
    plsc.subcore_barrier()

    def body(j, carry):
        pltpu.sync_copy(ones_v, cnt.at[dst_v.at[j]], add=True)
        return carry

    lax.fori_loop(0, 2 * NWIN, body, 0)
    plsc.subcore_barrier()
    pltpu.sync_copy(cnt.at[pl.ds(s * 2 * TROWS, 2 * TROWS)], out.at[c, s])


@functools.lru_cache(maxsize=None)
def _build_sc_kernels():
    mesh = plsc.VectorSubcoreMesh(core_axis_name="c", subcore_axis_name="s",
                                  num_cores=NC, num_subcores=NS)
    msg = pl.kernel(
        _sc_msg_body,
        jax.ShapeDtypeStruct((NC, NS, TROWS, F), jnp.float32),
        mesh=mesh,
        compiler_params=pltpu.CompilerParams(use_tc_tiling_on_sc=False),
        scratch_types=[
            pltpu.VMEM((NWIN, KB * CHUNK), jnp.int32),
            pltpu.VMEM((NWIN, KB * CHUNK), jnp.int32),
            pltpu.VMEM((NBUF, KB * CHUNK, F), jnp.float32),
            pltpu.SemaphoreType.DMA, pltpu.SemaphoreType.DMA,
            pltpu.SemaphoreType.DMA, pltpu.SemaphoreType.DMA,
            pltpu.VMEM_SHARED((P_ROWS, F), jnp.float32),
        ],
    )
    cnt = pl.kernel(
        _sc_cnt_body,
        jax.ShapeDtypeStruct((NC, NS, 2 * TROWS, CNT_W), jnp.float32),
        mesh=mesh,
        compiler_params=pltpu.CompilerParams(use_tc_tiling_on_sc=False),
        scratch_types=[
            pltpu.VMEM((2 * NWIN, KB * CHUNK), jnp.int32),
            pltpu.VMEM((KB * CHUNK, CNT_W), jnp.float32),
            pltpu.VMEM_SHARED((2 * P_ROWS, CNT_W), jnp.float32),
        ],
    )
    return msg, cnt


_BR = 1024


def _enc_body(x_ref, w_ref, b_ref, o_ref):
    o_ref[...] = (
        jnp.dot(x_ref[...], w_ref[...], preferred_element_type=jnp.float32)
        + b_ref[...]
    )


def _tc_enc(x, w, b):
    nblk = P_ROWS // _BR
    return pl.pallas_call(
        _enc_body,
        grid=(nblk,),
        in_specs=[
            pl.BlockSpec((_BR, 128), lambda i: (i, 0)),
            pl.BlockSpec((128, F), lambda i: (0, 0)),
            pl.BlockSpec((1, F), lambda i: (0, 0)),
        ],
        out_specs=pl.BlockSpec((_BR, F), lambda i: (i, 0)),
        out_shape=jax.ShapeDtypeStruct((P_ROWS, F), jnp.float32),
    )(x, w, b)


def _upd_body(p_ref, af_ref, ab_ref, cf_ref, cb_ref, wlf_ref, wlb_ref,
              wrf_ref, wrb_ref, bl_ref, o_ref):
    cf = cf_ref[0, :, 0:1] + cf_ref[1, :, 0:1]
    cb = cb_ref[0, :, 0:1] + cb_ref[1, :, 0:1]
    aggf = (af_ref[0] + af_ref[1]) * (0.5 / jnp.maximum(cf, 1.0))
    aggb = (ab_ref[0] + ab_ref[1]) * (0.5 / jnp.maximum(cb, 1.0))
    wr = 0.5 * (wrf_ref[...] + wrb_ref[...])
    acc = (
        jnp.dot(aggf, wlf_ref[...], preferred_element_type=jnp.float32)
        + jnp.dot(aggb, wlb_ref[...], preferred_element_type=jnp.float32)
        + jnp.dot(p_ref[...], wr, preferred_element_type=jnp.float32)
        + bl_ref[...]
    )
    o_ref[...] = jnp.maximum(acc, 0.0)


def _tc_update(p, accf, accb, cf, cb, wlf, wlb, wrf, wrb, bl):
    nblk = P_ROWS // _BR
    mat = lambda: pl.BlockSpec((F, F), lambda i: (0, 0))
    return pl.pallas_call(
        _upd_body,
        grid=(nblk,),
        in_specs=[
            pl.BlockSpec((_BR, F), lambda i: (i, 0)),
            pl.BlockSpec((NC, _BR, F), lambda i: (0, i, 0)),
            pl.BlockSpec((NC, _BR, F), lambda i: (0, i, 0)),
            pl.BlockSpec((NC, _BR, CNT_W), lambda i: (0, i, 0)),
            pl.BlockSpec((NC, _BR, CNT_W), lambda i: (0, i, 0)),
            mat(), mat(), mat(), mat(),
            pl.BlockSpec((1, F), lambda i: (0, 0)),
        ],
        out_specs=pl.BlockSpec((_BR, F), lambda i: (i, 0)),
        out_shape=jax.ShapeDtypeStruct((P_ROWS, F), jnp.float32),
    )(p, accf, accb, cf, cb, wlf, wlb, wrf, wrb, bl)


def _head_body(p3_ref, roh_ref, boh_ref, woh_ref, exp_ref,
               rt_ref, bt_ref, wt_ref, ew_ref, eb_ref,
               aggw_ref, aggb_ref, pw_ref, pb_ref,
               wf_ref, ww_ref, bw_ref, vf_ref, wv_ref, bv_ref,
               awv_ref, abv_ref, awo_ref, abo_ref,
               w1_ref, b1_ref, w2_ref, b2_ref, w3_ref, b3_ref,
               wa_ref, ba_ref, wb_ref, bb_ref,
               mp_ref, wip_ref, pe_ref, att_ref, ve_ref, te_ref):
    f32 = jnp.float32
    dot = functools.partial(jnp.dot, preferred_element_type=f32)
    rows = lax.broadcasted_iota(jnp.int32, (P_ROWS, 1), 0)
    pe = jnp.sum(jnp.where(rows < N_P, p3_ref[...], 0.0), axis=0,
                 keepdims=True) * (1.0 / N_P)
    role_e = dot(roh_ref[...], rt_ref[...])
    bat_e = dot(boh_ref[...], bt_ref[...])
    bowl_e = dot(woh_ref[...], wt_ref[...])
    exp_e = dot(exp_ref[...], ew_ref[...]) + eb_ref[...]
    t_i = lax.broadcasted_iota(jnp.int32, (2, 24), 0)
    r_i = lax.broadcasted_iota(jnp.int32, (2, 24), 1)
    sel = jnp.where((r_i >= 11 * t_i) & (r_i < 11 * t_i + 11),
                    f32(1.0 / 11.0), f32(0.0))
    sq = (
        dot(dot(sel, role_e), aggw_ref[0])
        + dot(dot(sel, bat_e), aggw_ref[1])
        + dot(dot(sel, bowl_e), aggw_ref[2])
        + dot(dot(sel, exp_e), aggw_ref[3])
        + aggb_ref[...]
    )
    te = dot(0.5 * jnp.sum(sq, axis=0, keepdims=True), pw_ref[...]) + pb_ref[...]
    wemb = dot(wf_ref[...], ww_ref[...]) + bw_ref[...]
    att = dot(dot(wemb, awv_ref[...]) + abv_ref[...], awo_ref[...]) + abo_ref[...]
    ve = dot(vf_ref[...], wv_ref[...]) + bv_ref[...]
    h = (
        dot(pe, w1_ref[0]) + dot(att, w1_ref[1]) + dot(ve, w1_ref[2])
        + dot(te, w1_ref[3]) + b1_ref[...]
    )
    h = jnp.maximum(h, 0.0)
    h = jnp.maximum(dot(h, w2_ref[...]) + b2_ref[...], 0.0)
    mp_ref[...] = dot(h, w3_ref[...]) + b3_ref[...]
    wip_ref[...] = (
        dot(jnp.maximum(dot(att, wa_ref[...]) + ba_ref[...], 0.0), wb_ref[...])
        + bb_ref[...]
    )
    pe_ref[...] = pe
    att_ref[...] = att
    ve_ref[...] = ve
    te_ref[...] = te


def _tc_head(args):
    outs = (
        jax.ShapeDtypeStruct((1, 1), jnp.float32),
        jax.ShapeDtypeStruct((1, 3), jnp.float32),
        jax.ShapeDtypeStruct((1, F), jnp.float32),
        jax.ShapeDtypeStruct((1, F), jnp.float32),
        jax.ShapeDtypeStruct((1, F), jnp.float32),
        jax.ShapeDtypeStruct((1, F), jnp.float32),
    )
    return pl.pallas_call(_head_body, out_shape=outs)(*args)


def _prep_idx(src, dst):
    pad = NE_PAD - NE
    src = jnp.concatenate([src.astype(jnp.int32), jnp.zeros((pad,), jnp.int32)])
    dst = jnp.concatenate(
        [dst.astype(jnp.int32), jnp.full((pad,), N_P, jnp.int32)])
    return (src.reshape(NW, NWIN, KB * CHUNK),
            dst.reshape(NW, NWIN, KB * CHUNK))


def kernel(x_player, x_venue, x_team, x_match, x_weather, ei_faced,
           ei_bowled_to, ei_played_at_pv, ei_plays_for, ei_played_at_mv,
           ei_had_weather, ei_played_in, weather_features, venue_features,
           role_idx, bat_idx, bowl_idx, exp_feats, params):
    f32 = jnp.float32
    _sc_msg, _sc_cnt = _build_sc_kernels()
    sf, df = _prep_idx(ei_faced[0], ei_faced[1])
    sb, db = _prep_idx(ei_bowled_to[0], ei_bowled_to[1])
    dst_cnt = jnp.concatenate([df, db + P_ROWS], axis=1)

    zeros_f = jnp.zeros((TROWS, F), f32)
    zeros_c = jnp.zeros((2 * TROWS, CNT_W), f32)
    ones_c = jnp.ones((KB * CHUNK, CNT_W), f32)

    cnt = _sc_cnt(dst_cnt, zeros_c, ones_c).reshape(NC, 2 * P_ROWS, CNT_W)
    cf, cb = cnt[:, :P_ROWS], cnt[:, P_ROWS:]

    ew, ebias = params["enc"]["player"]
    xp = jnp.pad(x_player, ((0, P_ROWS - N_P), (0, 0)))
    p = _tc_enc(xp, ew, ebias.reshape(1, F))

    for (wlf, blf, wrf), (wlb, blb, wrb) in (
            (layer[0], layer[1]) for layer in params["convs"]):
        accf = _sc_msg(p, sf, df, zeros_f).reshape(NC, P_ROWS, F)
        accb = _sc_msg(p, sb, db, zeros_f).reshape(NC, P_ROWS, F)
        p = _tc_update(p, accf, accb, cf, cb,
                       wlf, wlb, wrf, wrb,
                       (0.5 * (blf + blb)).reshape(1, F))

    sq = params["squad"]
    roh = jnp.pad(jax.nn.one_hot(role_idx, 5, dtype=f32).reshape(22, 5),
                  ((0, 2), (0, 3)))
    boh = jnp.pad(jax.nn.one_hot(bat_idx, 3, dtype=f32).reshape(22, 3),
                  ((0, 2), (0, 5)))
    woh = jnp.pad(jax.nn.one_hot(bowl_idx, 9, dtype=f32).reshape(22, 9),
                  ((0, 2), (0, 7)))
    expf = jnp.pad(exp_feats.reshape(22, 4), ((0, 2), (0, 4)))
    rt = jnp.pad(sq["role"], ((0, 3), (0, 0)))
    bt = jnp.pad(sq["bat"], ((0, 5), (0, 0)))
    wt = jnp.pad(sq["bowl"], ((0, 7), (0, 0)))
    eW, eb2 = sq["exp"]
    eWp = jnp.pad(eW, ((0, 4), (0, 0)))
    aggW, aggb = sq["agg"]
    pW, pb = params["proj"]
    Ww, bw = params["weather_enc"]
    Wve, bve = params["venue_enc"]
    at = params["attn"]
    (W1, b1), (W2, b2), (W3, b3) = params["mp"]
    (Wa, ba), (Wb2, bb2) = params["wip"]

    mp, wip, pe, att, ve, te = _tc_head((
        p, roh, boh, woh, expf,
        rt, bt, wt, eWp, eb2.reshape(1, 8),
        aggW.reshape(4, 8, 32), aggb.reshape(1, 32), pW, pb.reshape(1, F),
        weather_features.reshape(1, 16), Ww, bw.reshape(1, F),
        venue_features.reshape(1, 8), Wve, bve.reshape(1, F),
        at["Wv"][0], at["Wv"][1].reshape(1, F),
        at["Wo"][0], at["Wo"][1].reshape(1, F),
        W1.reshape(4, F, 2 * F), b1.reshape(1, 2 * F),
        W2, b2.reshape(1, F), W3, b3.reshape(1, 1),
        Wa, ba.reshape(1, 32), Wb2, bb2.reshape(1, 3),
    ))
    return (mp.reshape(1), wip.reshape(3), pe.reshape(F), att.reshape(F),
            ve.reshape(F), te.reshape(F))


# speedup vs baseline: 1.3229x; 1.3229x over previous
import functools

import jax
import jax.numpy as jnp
from jax import lax
from jax.experimental import pallas as pl
from jax.experimental.pallas import tpu as pltpu
from jax.experimental.pallas import tpu_sc as plsc

N_P = 10000
P_ROWS = 10240
NE = 320000
NC, NS = 2, 16
NW = NC * NS
CHUNK = 128
NE_PAD = 327680
CPT = NE_PAD // (NW * CHUNK)
TROWS = P_ROWS // NS
CNT_W = 16
F = 64


CPT2 = 2 * CPT


def _msg_loop(table, src_v, dst_v, rows0, rows1, sem0, sem1, acc):
    pltpu.async_copy(table.at[src_v.at[0]], rows0, sem0)
    pltpu.async_copy(table.at[src_v.at[1]], rows1, sem1)

    def body(j, carry):
        j0 = 2 * j
        j1 = j0 + 1
        pltpu.make_async_copy(table.at[src_v.at[j0]], rows0, sem0).wait()
        pltpu.sync_copy(rows0, acc.at[dst_v.at[j0]], add=True)

        @pl.when(j0 + 2 < CPT2)
        def _():
            pltpu.async_copy(table.at[src_v.at[j0 + 2]], rows0, sem0)

        pltpu.make_async_copy(table.at[src_v.at[j1]], rows1, sem1).wait()
        pltpu.sync_copy(rows1, acc.at[dst_v.at[j1]], add=True)

        @pl.when(j1 + 2 < CPT2)
        def _():
            pltpu.async_copy(table.at[src_v.at[j1 + 2]], rows1, sem1)

        return carry

    lax.fori_loop(0, CPT2 // 2, body, 0)


def _sc_msg_body(table, srcm, dstm, zeros, out, src_v, dst_v, rows0, rows1,
                 sem0, sem1, acc):
    c = lax.axis_index("c")
    s = lax.axis_index("s")
    pltpu.sync_copy(zeros, acc.at[pl.ds(s * TROWS, TROWS)])
    pltpu.sync_copy(srcm.at[c, s], src_v)
    pltpu.sync_copy(dstm.at[c, s], dst_v)
    plsc.subcore_barrier()
    _msg_loop(table, src_v, dst_v, rows0, rows1, sem0, sem1, acc)
    plsc.subcore_barrier()
    pltpu.sync_copy(acc.at[pl.ds(s * TROWS, TROWS)], out.at[c, s])


def _sc_msg1_body(table, srcm, dstm, zeros, zeros_c, ones, out, cnt_out,
                  src_v, dst_v, rows0, rows1, ones_v, sem0, sem1, acc,
                  cnt):
    c = lax.axis_index("c")
    s = lax.axis_index("s")
    pltpu.sync_copy(zeros, acc.at[pl.ds(s * TROWS, TROWS)])
    pltpu.sync_copy(zeros_c, cnt.at[pl.ds(s * CTROWS, CTROWS)])
    pltpu.sync_copy(srcm.at[c, s], src_v)
    pltpu.sync_copy(dstm.at[c, s], dst_v)
    pltpu.sync_copy(ones, ones_v)
    plsc.subcore_barrier()
    _msg_loop(table, src_v, dst_v, rows0, rows1, sem0, sem1, acc)

    def cbody(j, carry):
        pltpu.sync_copy(ones_v, cnt.at[dst_v.at[j]], add=True)
        return carry

    lax.fori_loop(0, CPT2, cbody, 0)
    plsc.subcore_barrier()
    pltpu.sync_copy(acc.at[pl.ds(s * TROWS, TROWS)], out.at[c, s])
    pltpu.sync_copy(cnt.at[pl.ds(s * CTROWS, CTROWS)], cnt_out.at[c, s])


CTROWS = P_ROWS // NS


@functools.lru_cache(maxsize=None)
def _build_sc_kernels():
    mesh = plsc.VectorSubcoreMesh(core_axis_name="c", subcore_axis_name="s",
                                  num_cores=NC, num_subcores=NS)
    common = [
        pltpu.VMEM((CPT2, CHUNK), jnp.int32),
        pltpu.VMEM((CPT2, CHUNK), jnp.int32),
        pltpu.VMEM((CHUNK, F), jnp.float32),
        pltpu.VMEM((CHUNK, F), jnp.float32),
    ]
    msg = pl.kernel(
        _sc_msg_body,
        jax.ShapeDtypeStruct((NC, NS, TROWS, F), jnp.float32),
        mesh=mesh,
        compiler_params=pltpu.CompilerParams(use_tc_tiling_on_sc=False),
        scratch_types=common + [
            pltpu.SemaphoreType.DMA, pltpu.SemaphoreType.DMA,
            pltpu.VMEM_SHARED((P_ROWS, F), jnp.float32),
        ],
    )
    msg1 = pl.kernel(
        _sc_msg1_body,
        (jax.ShapeDtypeStruct((NC, NS, TROWS, F), jnp.float32),
         jax.ShapeDtypeStruct((NC, NS, CTROWS, CNT_W), jnp.float32)),
        mesh=mesh,
        compiler_params=pltpu.CompilerParams(use_tc_tiling_on_sc=False),
        scratch_types=common + [
            pltpu.VMEM((CHUNK, CNT_W), jnp.float32),
            pltpu.SemaphoreType.DMA, pltpu.SemaphoreType.DMA,
            pltpu.VMEM_SHARED((P_ROWS, F), jnp.float32),
            pltpu.VMEM_SHARED((P_ROWS, CNT_W), jnp.float32),
        ],
    )
    return msg, msg1


_BR = 1024


def _enc_body(x_ref, w_ref, b_ref, o_ref):
    o_ref[...] = (
        jnp.dot(x_ref[...], w_ref[...], preferred_element_type=jnp.float32)
        + b_ref[...]
    )


def _tc_enc(x, w, b):
    nblk = P_ROWS // _BR
    return pl.pallas_call(
        _enc_body,
        grid=(nblk,),
        in_specs=[
            pl.BlockSpec((_BR, 128), lambda i: (i, 0)),
            pl.BlockSpec((128, F), lambda i: (0, 0)),
            pl.BlockSpec((1, F), lambda i: (0, 0)),
        ],
        out_specs=pl.BlockSpec((_BR, F), lambda i: (i, 0)),
        out_shape=jax.ShapeDtypeStruct((P_ROWS, F), jnp.float32),
    )(x, w, b)


def _upd_body(p_ref, af_ref, ab_ref, cf_ref, cb_ref, wlf_ref, wlb_ref,
              wrf_ref, wrb_ref, bl_ref, o_ref):
    aggf = af_ref[...] * (0.5 / jnp.maximum(cf_ref[:, 0:1], 1.0))
    aggb = ab_ref[...] * (0.5 / jnp.maximum(cb_ref[:, 0:1], 1.0))
    wr = 0.5 * (wrf_ref[...] + wrb_ref[...])
    acc = (
        jnp.dot(aggf, wlf_ref[...], preferred_element_type=jnp.float32)
        + jnp.dot(aggb, wlb_ref[...], preferred_element_type=jnp.float32)
        + jnp.dot(p_ref[...], wr, preferred_element_type=jnp.float32)
        + bl_ref[...]
    )
    o_ref[...] = jnp.maximum(acc, 0.0)


def _tc_update(p, accf, accb, cf, cb, wlf, wlb, wrf, wrb, bl):
    nblk = P_ROWS // _BR
    mat = lambda: pl.BlockSpec((F, F), lambda i: (0, 0))
    return pl.pallas_call(
        _upd_body,
        grid=(nblk,),
        in_specs=[
            pl.BlockSpec((_BR, F), lambda i: (i, 0)),
            pl.BlockSpec((_BR, F), lambda i: (i, 0)),
            pl.BlockSpec((_BR, F), lambda i: (i, 0)),
            pl.BlockSpec((_BR, CNT_W), lambda i: (i, 0)),
            pl.BlockSpec((_BR, CNT_W), lambda i: (i, 0)),
            mat(), mat(), mat(), mat(),
            pl.BlockSpec((1, F), lambda i: (0, 0)),
        ],
        out_specs=pl.BlockSpec((_BR, F), lambda i: (i, 0)),
        out_shape=jax.ShapeDtypeStruct((P_ROWS, F), jnp.float32),
    )(p, accf, accb, cf, cb, wlf, wlb, wrf, wrb, bl)


def _head_body(p3_ref, roh_ref, boh_ref, woh_ref, exp_ref,
               rt_ref, bt_ref, wt_ref, ew_ref, eb_ref,
               aggw_ref, aggb_ref, pw_ref, pb_ref,
               wf_ref, ww_ref, bw_ref, vf_ref, wv_ref, bv_ref,
               awv_ref, abv_ref, awo_ref, abo_ref,
               w1_ref, b1_ref, w2_ref, b2_ref, w3_ref, b3_ref,
               wa_ref, ba_ref, wb_ref, bb_ref,
               mp_ref, wip_ref, pe_ref, att_ref, ve_ref, te_ref):
    f32 = jnp.float32
    dot = functools.partial(jnp.dot, preferred_element_type=f32)
    rows = lax.broadcasted_iota(jnp.int32, (P_ROWS, 1), 0)
    pe = jnp.sum(jnp.where(rows < N_P, p3_ref[...], 0.0), axis=0,
                 keepdims=True) * (1.0 / N_P)
    role_e = dot(roh_ref[...], rt_ref[...])
    bat_e = dot(boh_ref[...], bt_ref[...])
    bowl_e = dot(woh_ref[...], wt_ref[...])
    exp_e = dot(exp_ref[...], ew_ref[...]) + eb_ref[...]
    t_i = lax.broadcasted_iota(jnp.int32, (2, 24), 0)
    r_i = lax.broadcasted_iota(jnp.int32, (2, 24), 1)
    sel = jnp.where((r_i >= 11 * t_i) & (r_i < 11 * t_i + 11),
                    f32(1.0 / 11.0), f32(0.0))
    sq = (
        dot(dot(sel, role_e), aggw_ref[0])
        + dot(dot(sel, bat_e), aggw_ref[1])
        + dot(dot(sel, bowl_e), aggw_ref[2])
        + dot(dot(sel, exp_e), aggw_ref[3])
        + aggb_ref[...]
    )
    te = dot(0.5 * jnp.sum(sq, axis=0, keepdims=True), pw_ref[...]) + pb_ref[...]
    wemb = dot(wf_ref[...], ww_ref[...]) + bw_ref[...]
    att = dot(dot(wemb, awv_ref[...]) + abv_ref[...], awo_ref[...]) + abo_ref[...]
    ve = dot(vf_ref[...], wv_ref[...]) + bv_ref[...]
    h = (
        dot(pe, w1_ref[0]) + dot(att, w1_ref[1]) + dot(ve, w1_ref[2])
        + dot(te, w1_ref[3]) + b1_ref[...]
    )
    h = jnp.maximum(h, 0.0)
    h = jnp.maximum(dot(h, w2_ref[...]) + b2_ref[...], 0.0)
    mp_ref[...] = dot(h, w3_ref[...]) + b3_ref[...]
    wip_ref[...] = (
        dot(jnp.maximum(dot(att, wa_ref[...]) + ba_ref[...], 0.0), wb_ref[...])
        + bb_ref[...]
    )
    pe_ref[...] = pe
    att_ref[...] = att
    ve_ref[...] = ve
    te_ref[...] = te


def _tc_head(args):
    outs = (
        jax.ShapeDtypeStruct((1, 1), jnp.float32),
        jax.ShapeDtypeStruct((1, 3), jnp.float32),
        jax.ShapeDtypeStruct((1, F), jnp.float32),
        jax.ShapeDtypeStruct((1, F), jnp.float32),
        jax.ShapeDtypeStruct((1, F), jnp.float32),
        jax.ShapeDtypeStruct((1, F), jnp.float32),
    )
    return pl.pallas_call(_head_body, out_shape=outs)(*args)


def _prep_idx(src, dst):
    pad = NE_PAD - NE
    src = jnp.concatenate([src.astype(jnp.int32), jnp.zeros((pad,), jnp.int32)])
    dst = jnp.concatenate(
        [dst.astype(jnp.int32), jnp.full((pad,), N_P, jnp.int32)])
    return src.reshape(NS, CPT2, CHUNK), dst.reshape(NS, CPT2, CHUNK)


def kernel(x_player, x_venue, x_team, x_match, x_weather, ei_faced,
           ei_bowled_to, ei_played_at_pv, ei_plays_for, ei_played_at_mv,
           ei_had_weather, ei_played_in, weather_features, venue_features,
           role_idx, bat_idx, bowl_idx, exp_feats, params):
    f32 = jnp.float32
    _sc_msg, _sc_msg1 = _build_sc_kernels()
    sf, df = _prep_idx(ei_faced[0], ei_faced[1])
    sb, db = _prep_idx(ei_bowled_to[0], ei_bowled_to[1])
    srcm = jnp.stack([sf, sb])
    dstm = jnp.stack([df, db])

    zeros_f = jnp.zeros((TROWS, F), f32)
    zeros_c = jnp.zeros((CTROWS, CNT_W), f32)
    ones_c = jnp.ones((CHUNK, CNT_W), f32)

    ew, ebias = params["enc"]["player"]
    xp = jnp.pad(x_player, ((0, P_ROWS - N_P), (0, 0)))
    p = _tc_enc(xp, ew, ebias.reshape(1, F))

    cf = cb = None
    for li, ((wlf, blf, wrf), (wlb, blb, wrb)) in enumerate(
            (layer[0], layer[1]) for layer in params["convs"]):
        if li == 0:
            acc, cnt = _sc_msg1(p, srcm, dstm, zeros_f, zeros_c, ones_c)
            cf = cnt[0].reshape(P_ROWS, CNT_W)
            cb = cnt[1].reshape(P_ROWS, CNT_W)
        else:
            acc = _sc_msg(p, srcm, dstm, zeros_f)
        p = _tc_update(p, acc[0].reshape(P_ROWS, F), acc[1].reshape(P_ROWS, F),
                       cf, cb, wlf, wlb, wrf, wrb,
                       (0.5 * (blf + blb)).reshape(1, F))

    sq = params["squad"]
    roh = jnp.pad(jax.nn.one_hot(role_idx, 5, dtype=f32).reshape(22, 5),
                  ((0, 2), (0, 3)))
    boh = jnp.pad(jax.nn.one_hot(bat_idx, 3, dtype=f32).reshape(22, 3),
                  ((0, 2), (0, 5)))
    woh = jnp.pad(jax.nn.one_hot(bowl_idx, 9, dtype=f32).reshape(22, 9),
                  ((0, 2), (0, 7)))
    expf = jnp.pad(exp_feats.reshape(22, 4), ((0, 2), (0, 4)))
    rt = jnp.pad(sq["role"], ((0, 3), (0, 0)))
    bt = jnp.pad(sq["bat"], ((0, 5), (0, 0)))
    wt = jnp.pad(sq["bowl"], ((0, 7), (0, 0)))
    eW, eb2 = sq["exp"]
    eWp = jnp.pad(eW, ((0, 4), (0, 0)))
    aggW, aggb = sq["agg"]
    pW, pb = params["proj"]
    Ww, bw = params["weather_enc"]
    Wve, bve = params["venue_enc"]
    at = params["attn"]
    (W1, b1), (W2, b2), (W3, b3) = params["mp"]
    (Wa, ba), (Wb2, bb2) = params["wip"]

    mp, wip, pe, att, ve, te = _tc_head((
        p, roh, boh, woh, expf,
        rt, bt, wt, eWp, eb2.reshape(1, 8),
        aggW.reshape(4, 8, 32), aggb.reshape(1, 32), pW, pb.reshape(1, F),
        weather_features.reshape(1, 16), Ww, bw.reshape(1, F),
        venue_features.reshape(1, 8), Wve, bve.reshape(1, F),
        at["Wv"][0], at["Wv"][1].reshape(1, F),
        at["Wo"][0], at["Wo"][1].reshape(1, F),
        W1.reshape(4, F, 2 * F), b1.reshape(1, 2 * F),
        W2, b2.reshape(1, F), W3, b3.reshape(1, 1),
        Wa, ba.reshape(1, 32), Wb2, bb2.reshape(1, 3),
    ))
    return (mp.reshape(1), wip.reshape(3), pe.reshape(F), att.reshape(F),
            ve.reshape(F), te.reshape(F))


# speedup vs baseline: 3.3149x; 2.5057x over previous
import functools

import jax
import jax.numpy as jnp
from jax import lax
from jax.experimental import pallas as pl
from jax.experimental.pallas import tpu as pltpu
from jax.experimental.pallas import tpu_sc as plsc

N_P = 10000
P_ROWS = 10240
NE = 320000
NC, NS = 2, 16
NW = NC * NS
CHUNK = 128
NE_PAD = 327680
CPT = NE_PAD // (NW * CHUNK)
TROWS = P_ROWS // NS
CNT_W = 16
F = 64


CPT2 = 2 * CPT


def _msg_loop(table, src_v, dst_v, rows0, rows1, sem0, sem1, acc):
    pltpu.async_copy(table.at[src_v.at[0]], rows0, sem0)
    pltpu.async_copy(table.at[src_v.at[1]], rows1, sem1)

    def body(j, carry):
        j0 = 2 * j
        j1 = j0 + 1
        pltpu.make_async_copy(table.at[src_v.at[j0]], rows0, sem0).wait()
        pltpu.sync_copy(rows0, acc.at[dst_v.at[j0]], add=True)

        @pl.when(j0 + 2 < CPT2)
        def _():
            pltpu.async_copy(table.at[src_v.at[j0 + 2]], rows0, sem0)

        pltpu.make_async_copy(table.at[src_v.at[j1]], rows1, sem1).wait()
        pltpu.sync_copy(rows1, acc.at[dst_v.at[j1]], add=True)

        @pl.when(j1 + 2 < CPT2)
        def _():
            pltpu.async_copy(table.at[src_v.at[j1 + 2]], rows1, sem1)

        return carry

    lax.fori_loop(0, CPT2 // 2, body, 0)


def _sc_msg_body(table, srcm, dstm, zeros, out, src_v, dst_v, rows0, rows1,
                 sem0, sem1, acc):
    c = lax.axis_index("c")
    s = lax.axis_index("s")
    pltpu.sync_copy(zeros, acc.at[pl.ds(s * TROWS, TROWS)])
    pltpu.sync_copy(srcm.at[c, s], src_v)
    pltpu.sync_copy(dstm.at[c, s], dst_v)
    plsc.subcore_barrier()
    _msg_loop(table, src_v, dst_v, rows0, rows1, sem0, sem1, acc)
    plsc.subcore_barrier()
    pltpu.sync_copy(acc.at[pl.ds(s * TROWS, TROWS)], out.at[c, s])


def _sc_msg1_body(table, srcm, dstm, zeros, zeros_c, ones, out, cnt_out,
                  src_v, dst_v, rows0, rows1, ones_v, sem0, sem1, acc,
                  cnt):
    c = lax.axis_index("c")
    s = lax.axis_index("s")
    pltpu.sync_copy(zeros, acc.at[pl.ds(s * TROWS, TROWS)])
    pltpu.sync_copy(zeros_c, cnt.at[pl.ds(s * CTROWS, CTROWS)])
    pltpu.sync_copy(srcm.at[c, s], src_v)
    pltpu.sync_copy(dstm.at[c, s], dst_v)
    pltpu.sync_copy(ones, ones_v)
    plsc.subcore_barrier()
    _msg_loop(table, src_v, dst_v, rows0, rows1, sem0, sem1, acc)

    def cbody(j, carry):
        pltpu.sync_copy(ones_v, cnt.at[dst_v.at[j]], add=True)
        return carry

    lax.fori_loop(0, CPT2, cbody, 0)
    plsc.subcore_barrier()
    pltpu.sync_copy(acc.at[pl.ds(s * TROWS, TROWS)], out.at[c, s])
    pltpu.sync_copy(cnt.at[pl.ds(s * CTROWS, CTROWS)], cnt_out.at[c, s])


CTROWS = P_ROWS // NS


@functools.lru_cache(maxsize=None)
def _build_sc_kernels():
    mesh = plsc.VectorSubcoreMesh(core_axis_name="c", subcore_axis_name="s",
                                  num_cores=NC, num_subcores=NS)
    common = [
        pltpu.VMEM((CPT2, CHUNK), jnp.int32),
        pltpu.VMEM((CPT2, CHUNK), jnp.int32),
        pltpu.VMEM((CHUNK, F), jnp.float32),
        pltpu.VMEM((CHUNK, F), jnp.float32),
    ]
    msg = pl.kernel(
        _sc_msg_body,
        jax.ShapeDtypeStruct((NC, NS, TROWS, F), jnp.float32),
        mesh=mesh,
        compiler_params=pltpu.CompilerParams(use_tc_tiling_on_sc=False),
        scratch_types=common + [
            pltpu.SemaphoreType.DMA, pltpu.SemaphoreType.DMA,
            pltpu.VMEM_SHARED((P_ROWS, F), jnp.float32),
        ],
    )
    msg1 = pl.kernel(
        _sc_msg1_body,
        (jax.ShapeDtypeStruct((NC, NS, TROWS, F), jnp.float32),
         jax.ShapeDtypeStruct((NC, NS, CTROWS, CNT_W), jnp.float32)),
        mesh=mesh,
        compiler_params=pltpu.CompilerParams(use_tc_tiling_on_sc=False),
        scratch_types=common + [
            pltpu.VMEM((CHUNK, CNT_W), jnp.float32),
            pltpu.SemaphoreType.DMA, pltpu.SemaphoreType.DMA,
            pltpu.VMEM_SHARED((P_ROWS, F), jnp.float32),
            pltpu.VMEM_SHARED((P_ROWS, CNT_W), jnp.float32),
        ],
    )
    return msg, msg1


_BR = 1024


def _enc_body(x_ref, w_ref, b_ref, o_ref):
    o_ref[...] = (
        jnp.dot(x_ref[...], w_ref[...], preferred_element_type=jnp.float32)
        + b_ref[...]
    )


def _tc_enc(x, w, b):
    nblk = P_ROWS // _BR
    return pl.pallas_call(
        _enc_body,
        grid=(nblk,),
        in_specs=[
            pl.BlockSpec((_BR, 128), lambda i: (i, 0)),
            pl.BlockSpec((128, F), lambda i: (0, 0)),
            pl.BlockSpec((1, F), lambda i: (0, 0)),
        ],
        out_specs=pl.BlockSpec((_BR, F), lambda i: (i, 0)),
        out_shape=jax.ShapeDtypeStruct((P_ROWS, F), jnp.float32),
    )(x, w, b)


def _upd_body(p_ref, af_ref, ab_ref, cf_ref, cb_ref, wlf_ref, wlb_ref,
              wrf_ref, wrb_ref, bl_ref, o_ref):
    aggf = af_ref[0, 0] * (0.5 / jnp.maximum(cf_ref[0, 0, :, 0:1], 1.0))
    aggb = ab_ref[0, 0] * (0.5 / jnp.maximum(cb_ref[0, 0, :, 0:1], 1.0))
    wr = 0.5 * (wrf_ref[...] + wrb_ref[...])
    acc = (
        jnp.dot(aggf, wlf_ref[...], preferred_element_type=jnp.float32)
        + jnp.dot(aggb, wlb_ref[...], preferred_element_type=jnp.float32)
        + jnp.dot(p_ref[...], wr, preferred_element_type=jnp.float32)
        + bl_ref[...]
    )
    o_ref[...] = jnp.maximum(acc, 0.0)


def _tc_update(p, acc, cnt, wlf, wlb, wrf, wrb, bl):
    mat = lambda: pl.BlockSpec((F, F), lambda i: (0, 0))
    return pl.pallas_call(
        _upd_body,
        grid=(NS,),
        in_specs=[
            pl.BlockSpec((TROWS, F), lambda i: (i, 0)),
            pl.BlockSpec((1, 1, TROWS, F), lambda i: (0, i, 0, 0)),
            pl.BlockSpec((1, 1, TROWS, F), lambda i: (1, i, 0, 0)),
            pl.BlockSpec((1, 1, CTROWS, CNT_W), lambda i: (0, i, 0, 0)),
            pl.BlockSpec((1, 1, CTROWS, CNT_W), lambda i: (1, i, 0, 0)),
            mat(), mat(), mat(), mat(),
            pl.BlockSpec((1, F), lambda i: (0, 0)),
        ],
        out_specs=pl.BlockSpec((TROWS, F), lambda i: (i, 0)),
        out_shape=jax.ShapeDtypeStruct((P_ROWS, F), jnp.float32),
    )(p, acc, acc, cnt, cnt, wlf, wlb, wrf, wrb, bl)


def _head_body(p3_ref, roh_ref, boh_ref, woh_ref, exp_ref,
               rt_ref, bt_ref, wt_ref, ew_ref, eb_ref,
               aggw_ref, aggb_ref, pw_ref, pb_ref,
               wf_ref, ww_ref, bw_ref, vf_ref, wv_ref, bv_ref,
               awv_ref, abv_ref, awo_ref, abo_ref,
               w1_ref, b1_ref, w2_ref, b2_ref, w3_ref, b3_ref,
               wa_ref, ba_ref, wb_ref, bb_ref,
               mp_ref, wip_ref, pe_ref, att_ref, ve_ref, te_ref):
    f32 = jnp.float32
    dot = functools.partial(jnp.dot, preferred_element_type=f32)
    rows = lax.broadcasted_iota(jnp.int32, (P_ROWS, 1), 0)
    pe = jnp.sum(jnp.where(rows < N_P, p3_ref[...], 0.0), axis=0,
                 keepdims=True) * (1.0 / N_P)
    role_e = dot(roh_ref[...], rt_ref[...])
    bat_e = dot(boh_ref[...], bt_ref[...])
    bowl_e = dot(woh_ref[...], wt_ref[...])
    exp_e = dot(exp_ref[...], ew_ref[...]) + eb_ref[...]
    t_i = lax.broadcasted_iota(jnp.int32, (2, 24), 0)
    r_i = lax.broadcasted_iota(jnp.int32, (2, 24), 1)
    sel = jnp.where((r_i >= 11 * t_i) & (r_i < 11 * t_i + 11),
                    f32(1.0 / 11.0), f32(0.0))
    sq = (
        dot(dot(sel, role_e), aggw_ref[0])
        + dot(dot(sel, bat_e), aggw_ref[1])
        + dot(dot(sel, bowl_e), aggw_ref[2])
        + dot(dot(sel, exp_e), aggw_ref[3])
        + aggb_ref[...]
    )
    te = dot(0.5 * jnp.sum(sq, axis=0, keepdims=True), pw_ref[...]) + pb_ref[...]
    wemb = dot(wf_ref[...], ww_ref[...]) + bw_ref[...]
    att = dot(dot(wemb, awv_ref[...]) + abv_ref[...], awo_ref[...]) + abo_ref[...]
    ve = dot(vf_ref[...], wv_ref[...]) + bv_ref[...]
    h = (
        dot(pe, w1_ref[0]) + dot(att, w1_ref[1]) + dot(ve, w1_ref[2])
        + dot(te, w1_ref[3]) + b1_ref[...]
    )
    h = jnp.maximum(h, 0.0)
    h = jnp.maximum(dot(h, w2_ref[...]) + b2_ref[...], 0.0)
    mp_ref[...] = dot(h, w3_ref[...]) + b3_ref[...]
    wip_ref[...] = (
        dot(jnp.maximum(dot(att, wa_ref[...]) + ba_ref[...], 0.0), wb_ref[...])
        + bb_ref[...]
    )
    pe_ref[...] = pe
    att_ref[...] = att
    ve_ref[...] = ve
    te_ref[...] = te


def _tc_head(args):
    outs = (
        jax.ShapeDtypeStruct((1, 1), jnp.float32),
        jax.ShapeDtypeStruct((1, 3), jnp.float32),
        jax.ShapeDtypeStruct((1, F), jnp.float32),
        jax.ShapeDtypeStruct((1, F), jnp.float32),
        jax.ShapeDtypeStruct((1, F), jnp.float32),
        jax.ShapeDtypeStruct((1, F), jnp.float32),
    )
    return pl.pallas_call(_head_body, out_shape=outs)(*args)


def _prep_idx(src, dst):
    pad = NE_PAD - NE
    spread = jnp.arange(pad, dtype=jnp.int32)
    src = jnp.concatenate([src.astype(jnp.int32), spread % N_P])
    dst = jnp.concatenate(
        [dst.astype(jnp.int32), N_P + spread % (P_ROWS - N_P)])
    return src.reshape(NS, CPT2, CHUNK), dst.reshape(NS, CPT2, CHUNK)


def kernel(x_player, x_venue, x_team, x_match, x_weather, ei_faced,
           ei_bowled_to, ei_played_at_pv, ei_plays_for, ei_played_at_mv,
           ei_had_weather, ei_played_in, weather_features, venue_features,
           role_idx, bat_idx, bowl_idx, exp_feats, params):
    f32 = jnp.float32
    _sc_msg, _sc_msg1 = _build_sc_kernels()
    sf, df = _prep_idx(ei_faced[0], ei_faced[1])
    sb, db = _prep_idx(ei_bowled_to[0], ei_bowled_to[1])
    srcm = jnp.stack([sf, sb])
    dstm = jnp.stack([df, db])

    zeros_f = jnp.zeros((TROWS, F), f32)
    zeros_c = jnp.zeros((CTROWS, CNT_W), f32)
    ones_c = jnp.ones((CHUNK, CNT_W), f32)

    ew, ebias = params["enc"]["player"]
    xp = jnp.pad(x_player, ((0, P_ROWS - N_P), (0, 0)))
    p = _tc_enc(xp, ew, ebias.reshape(1, F))

    cnt = None
    for li, ((wlf, blf, wrf), (wlb, blb, wrb)) in enumerate(
            (layer[0], layer[1]) for layer in params["convs"]):
        if li == 0:
            acc, cnt = _sc_msg1(p, srcm, dstm, zeros_f, zeros_c, ones_c)
        else:
            acc = _sc_msg(p, srcm, dstm, zeros_f)
        p = _tc_update(p, acc, cnt, wlf, wlb, wrf, wrb,
                       (0.5 * (blf + blb)).reshape(1, F))

    sq = params["squad"]
    roh = jnp.pad(jax.nn.one_hot(role_idx, 5, dtype=f32).reshape(22, 5),
                  ((0, 2), (0, 3)))
    boh = jnp.pad(jax.nn.one_hot(bat_idx, 3, dtype=f32).reshape(22, 3),
                  ((0, 2), (0, 5)))
    woh = jnp.pad(jax.nn.one_hot(bowl_idx, 9, dtype=f32).reshape(22, 9),
                  ((0, 2), (0, 7)))
    expf = jnp.pad(exp_feats.reshape(22, 4), ((0, 2), (0, 4)))
    rt = jnp.pad(sq["role"], ((0, 3), (0, 0)))
    bt = jnp.pad(sq["bat"], ((0, 5), (0, 0)))
    wt = jnp.pad(sq["bowl"], ((0, 7), (0, 0)))
    eW, eb2 = sq["exp"]
    eWp = jnp.pad(eW, ((0, 4), (0, 0)))
    aggW, aggb = sq["agg"]
    pW, pb = params["proj"]
    Ww, bw = params["weather_enc"]
    Wve, bve = params["venue_enc"]
    at = params["attn"]
    (W1, b1), (W2, b2), (W3, b3) = params["mp"]
    (Wa, ba), (Wb2, bb2) = params["wip"]

    mp, wip, pe, att, ve, te = _tc_head((
        p, roh, boh, woh, expf,
        rt, bt, wt, eWp, eb2.reshape(1, 8),
        aggW.reshape(4, 8, 32), aggb.reshape(1, 32), pW, pb.reshape(1, F),
        weather_features.reshape(1, 16), Ww, bw.reshape(1, F),
        venue_features.reshape(1, 8), Wve, bve.reshape(1, F),
        at["Wv"][0], at["Wv"][1].reshape(1, F),
        at["Wo"][0], at["Wo"][1].reshape(1, F),
        W1.reshape(4, F, 2 * F), b1.reshape(1, 2 * F),
        W2, b2.reshape(1, F), W3, b3.reshape(1, 1),
        Wa, ba.reshape(1, 32), Wb2, bb2.reshape(1, 3),
    ))
    return (mp.reshape(1), wip.reshape(3), pe.reshape(F), att.reshape(F),
            ve.reshape(F), te.reshape(F))


# speedup vs baseline: 3.5621x; 1.0746x over previous
import functools

import jax
import jax.numpy as jnp
from jax import lax
from jax.experimental import pallas as pl
from jax.experimental.pallas import tpu as pltpu
from jax.experimental.pallas import tpu_sc as plsc

N_P = 10000
P_ROWS = 10240
NE = 320000
NC, NS = 2, 16
NW = NC * NS
CHUNK = 128
NE_PAD = 327680
CPT = NE_PAD // (NW * CHUNK)
TROWS = P_ROWS // NS
CNT_W = 16
F = 64


CPT2 = 2 * CPT


NBUF = 4


def _msg_loop(table, src_v, dst_v, rows, gsems, ssems, acc):
    def body(j, carry):
        for b in range(NBUF):
            k = NBUF * j + b

            @pl.when(j > 0)
            def _():
                pltpu.make_async_copy(
                    rows.at[b], acc.at[dst_v.at[k - NBUF]], ssems[b]).wait()

            pltpu.async_copy(table.at[src_v.at[k]], rows.at[b], gsems[b])
        for b in range(NBUF):
            k = NBUF * j + b
            pltpu.make_async_copy(
                table.at[src_v.at[k]], rows.at[b], gsems[b]).wait()
            pltpu.async_copy(rows.at[b], acc.at[dst_v.at[k]], ssems[b],
                             add=True)
        return carry

    lax.fori_loop(0, CPT2 // NBUF, body, 0)
    for b in range(NBUF):
        pltpu.make_async_copy(
            rows.at[b], acc.at[dst_v.at[CPT2 - NBUF + b]], ssems[b]).wait()


def _sc_msg_body(table, srcm, dstm, zeros, out, src_v, dst_v, rows,
                 g0, g1, g2, g3, s0, s1, s2, s3, acc):
    c = lax.axis_index("c")
    s = lax.axis_index("s")
    pltpu.sync_copy(zeros, acc.at[pl.ds(s * TROWS, TROWS)])
    pltpu.sync_copy(srcm.at[c, s], src_v)
    pltpu.sync_copy(dstm.at[c, s], dst_v)
    plsc.subcore_barrier()
    _msg_loop(table, src_v, dst_v, rows, (g0, g1, g2, g3), (s0, s1, s2, s3),
              acc)
    plsc.subcore_barrier()
    pltpu.sync_copy(acc.at[pl.ds(s * TROWS, TROWS)], out.at[c, s])


def _sc_msg1_body(table, srcm, dstm, zeros, zeros_c, ones, out, cnt_out,
                  src_v, dst_v, rows, ones_v, g0, g1, g2, g3, s0, s1, s2, s3,
                  acc, cnt):
    c = lax.axis_index("c")
    s = lax.axis_index("s")
    pltpu.sync_copy(zeros, acc.at[pl.ds(s * TROWS, TROWS)])
    pltpu.sync_copy(zeros_c, cnt.at[pl.ds(s * CTROWS, CTROWS)])
    pltpu.sync_copy(srcm.at[c, s], src_v)
    pltpu.sync_copy(dstm.at[c, s], dst_v)
    pltpu.sync_copy(ones, ones_v)
    plsc.subcore_barrier()
    _msg_loop(table, src_v, dst_v, rows, (g0, g1, g2, g3), (s0, s1, s2, s3),
              acc)

    def cbody(j, carry):
        pltpu.sync_copy(ones_v, cnt.at[dst_v.at[j]], add=True)
        return carry

    lax.fori_loop(0, CPT2, cbody, 0)
    plsc.subcore_barrier()
    pltpu.sync_copy(acc.at[pl.ds(s * TROWS, TROWS)], out.at[c, s])
    pltpu.sync_copy(cnt.at[pl.ds(s * CTROWS, CTROWS)], cnt_out.at[c, s])


CTROWS = P_ROWS // NS


@functools.lru_cache(maxsize=None)
def _build_sc_kernels():
    mesh = plsc.VectorSubcoreMesh(core_axis_name="c", subcore_axis_name="s",
                                  num_cores=NC, num_subcores=NS)
    common = [
        pltpu.VMEM((CPT2, CHUNK), jnp.int32),
        pltpu.VMEM((CPT2, CHUNK), jnp.int32),
        pltpu.VMEM((NBUF, CHUNK, F), jnp.float32),
    ]
    sems = [pltpu.SemaphoreType.DMA] * (2 * NBUF)
    msg = pl.kernel(
        _sc_msg_body,
        jax.ShapeDtypeStruct((NC, NS, TROWS, F), jnp.float32),
        mesh=mesh,
        compiler_params=pltpu.CompilerParams(use_tc_tiling_on_sc=False),
        scratch_types=common + sems + [
            pltpu.VMEM_SHARED((P_ROWS, F), jnp.float32),
        ],
    )
    msg1 = pl.kernel(
        _sc_msg1_body,
        (jax.ShapeDtypeStruct((NC, NS, TROWS, F), jnp.float32),
         jax.ShapeDtypeStruct((NC, NS, CTROWS, CNT_W), jnp.float32)),
        mesh=mesh,
        compiler_params=pltpu.CompilerParams(use_tc_tiling_on_sc=False),
        scratch_types=common + [
            pltpu.VMEM((CHUNK, CNT_W), jnp.float32),
        ] + sems + [
            pltpu.VMEM_SHARED((P_ROWS, F), jnp.float32),
            pltpu.VMEM_SHARED((P_ROWS, CNT_W), jnp.float32),
        ],
    )
    return msg, msg1


_BR = 1024


def _enc_body(x_ref, w_ref, b_ref, o_ref):
    o_ref[...] = (
        jnp.dot(x_ref[...], w_ref[...], preferred_element_type=jnp.float32)
        + b_ref[...]
    )


def _tc_enc(x, w, b):
    nblk = P_ROWS // _BR
    return pl.pallas_call(
        _enc_body,
        grid=(nblk,),
        in_specs=[
            pl.BlockSpec((_BR, 128), lambda i: (i, 0)),
            pl.BlockSpec((128, F), lambda i: (0, 0)),
            pl.BlockSpec((1, F), lambda i: (0, 0)),
        ],
        out_specs=pl.BlockSpec((_BR, F), lambda i: (i, 0)),
        out_shape=jax.ShapeDtypeStruct((P_ROWS, F), jnp.float32),
    )(x, w, b)


def _upd_body(p_ref, af_ref, ab_ref, cf_ref, cb_ref, wlf_ref, wlb_ref,
              wrf_ref, wrb_ref, bl_ref, o_ref):
    aggf = af_ref[0, 0] * (0.5 / jnp.maximum(cf_ref[0, 0, :, 0:1], 1.0))
    aggb = ab_ref[0, 0] * (0.5 / jnp.maximum(cb_ref[0, 0, :, 0:1], 1.0))
    wr = 0.5 * (wrf_ref[...] + wrb_ref[...])
    acc = (
        jnp.dot(aggf, wlf_ref[...], preferred_element_type=jnp.float32)
        + jnp.dot(aggb, wlb_ref[...], preferred_element_type=jnp.float32)
        + jnp.dot(p_ref[...], wr, preferred_element_type=jnp.float32)
        + bl_ref[...]
    )
    o_ref[...] = jnp.maximum(acc, 0.0)


def _tc_update(p, acc, cnt, wlf, wlb, wrf, wrb, bl):
    mat = lambda: pl.BlockSpec((F, F), lambda i: (0, 0))
    return pl.pallas_call(
        _upd_body,
        grid=(NS,),
        in_specs=[
            pl.BlockSpec((TROWS, F), lambda i: (i, 0)),
            pl.BlockSpec((1, 1, TROWS, F), lambda i: (0, i, 0, 0)),
            pl.BlockSpec((1, 1, TROWS, F), lambda i: (1, i, 0, 0)),
            pl.BlockSpec((1, 1, CTROWS, CNT_W), lambda i: (0, i, 0, 0)),
            pl.BlockSpec((1, 1, CTROWS, CNT_W), lambda i: (1, i, 0, 0)),
            mat(), mat(), mat(), mat(),
            pl.BlockSpec((1, F), lambda i: (0, 0)),
        ],
        out_specs=pl.BlockSpec((TROWS, F), lambda i: (i, 0)),
        out_shape=jax.ShapeDtypeStruct((P_ROWS, F), jnp.float32),
    )(p, acc, acc, cnt, cnt, wlf, wlb, wrf, wrb, bl)


def _head_body(p3_ref, roh_ref, boh_ref, woh_ref, exp_ref,
               rt_ref, bt_ref, wt_ref, ew_ref, eb_ref,
               aggw_ref, aggb_ref, pw_ref, pb_ref,
               wf_ref, ww_ref, bw_ref, vf_ref, wv_ref, bv_ref,
               awv_ref, abv_ref, awo_ref, abo_ref,
               w1_ref, b1_ref, w2_ref, b2_ref, w3_ref, b3_ref,
               wa_ref, ba_ref, wb_ref, bb_ref,
               mp_ref, wip_ref, pe_ref, att_ref, ve_ref, te_ref):
    f32 = jnp.float32
    dot = functools.partial(jnp.dot, preferred_element_type=f32)
    rows = lax.broadcasted_iota(jnp.int32, (P_ROWS, 1), 0)
    pe = jnp.sum(jnp.where(rows < N_P, p3_ref[...], 0.0), axis=0,
                 keepdims=True) * (1.0 / N_P)
    role_e = dot(roh_ref[...], rt_ref[...])
    bat_e = dot(boh_ref[...], bt_ref[...])
    bowl_e = dot(woh_ref[...], wt_ref[...])
    exp_e = dot(exp_ref[...], ew_ref[...]) + eb_ref[...]
    t_i = lax.broadcasted_iota(jnp.int32, (2, 24), 0)
    r_i = lax.broadcasted_iota(jnp.int32, (2, 24), 1)
    sel = jnp.where((r_i >= 11 * t_i) & (r_i < 11 * t_i + 11),
                    f32(1.0 / 11.0), f32(0.0))
    sq = (
        dot(dot(sel, role_e), aggw_ref[0])
        + dot(dot(sel, bat_e), aggw_ref[1])
        + dot(dot(sel, bowl_e), aggw_ref[2])
        + dot(dot(sel, exp_e), aggw_ref[3])
        + aggb_ref[...]
    )
    te = dot(0.5 * jnp.sum(sq, axis=0, keepdims=True), pw_ref[...]) + pb_ref[...]
    wemb = dot(wf_ref[...], ww_ref[...]) + bw_ref[...]
    att = dot(dot(wemb, awv_ref[...]) + abv_ref[...], awo_ref[...]) + abo_ref[...]
    ve = dot(vf_ref[...], wv_ref[...]) + bv_ref[...]
    h = (
        dot(pe, w1_ref[0]) + dot(att, w1_ref[1]) + dot(ve, w1_ref[2])
        + dot(te, w1_ref[3]) + b1_ref[...]
    )
    h = jnp.maximum(h, 0.0)
    h = jnp.maximum(dot(h, w2_ref[...]) + b2_ref[...], 0.0)
    mp_ref[...] = dot(h, w3_ref[...]) + b3_ref[...]
    wip_ref[...] = (
        dot(jnp.maximum(dot(att, wa_ref[...]) + ba_ref[...], 0.0), wb_ref[...])
        + bb_ref[...]
    )
    pe_ref[...] = pe
    att_ref[...] = att
    ve_ref[...] = ve
    te_ref[...] = te


def _tc_head(args):
    outs = (
        jax.ShapeDtypeStruct((1, 1), jnp.float32),
        jax.ShapeDtypeStruct((1, 3), jnp.float32),
        jax.ShapeDtypeStruct((1, F), jnp.float32),
        jax.ShapeDtypeStruct((1, F), jnp.float32),
        jax.ShapeDtypeStruct((1, F), jnp.float32),
        jax.ShapeDtypeStruct((1, F), jnp.float32),
    )
    return pl.pallas_call(_head_body, out_shape=outs)(*args)


def _prep_idx(src, dst):
    pad = NE_PAD - NE
    spread = jnp.arange(pad, dtype=jnp.int32)
    src = jnp.concatenate([src.astype(jnp.int32), spread % N_P])
    dst = jnp.concatenate(
        [dst.astype(jnp.int32), N_P + spread % (P_ROWS - N_P)])
    return src.reshape(NS, CPT2, CHUNK), dst.reshape(NS, CPT2, CHUNK)


def kernel(x_player, x_venue, x_team, x_match, x_weather, ei_faced,
           ei_bowled_to, ei_played_at_pv, ei_plays_for, ei_played_at_mv,
           ei_had_weather, ei_played_in, weather_features, venue_features,
           role_idx, bat_idx, bowl_idx, exp_feats, params):
    f32 = jnp.float32
    _sc_msg, _sc_msg1 = _build_sc_kernels()
    sf, df = _prep_idx(ei_faced[0], ei_faced[1])
    sb, db = _prep_idx(ei_bowled_to[0], ei_bowled_to[1])
    srcm = jnp.stack([sf, sb])
    dstm = jnp.stack([df, db])

    zeros_f = jnp.zeros((TROWS, F), f32)
    zeros_c = jnp.zeros((CTROWS, CNT_W), f32)
    ones_c = jnp.ones((CHUNK, CNT_W), f32)

    ew, ebias = params["enc"]["player"]
    xp = jnp.pad(x_player, ((0, P_ROWS - N_P), (0, 0)))
    p = _tc_enc(xp, ew, ebias.reshape(1, F))

    cnt = None
    for li, ((wlf, blf, wrf), (wlb, blb, wrb)) in enumerate(
            (layer[0], layer[1]) for layer in params["convs"]):
        if li == 0:
            acc, cnt = _sc_msg1(p, srcm, dstm, zeros_f, zeros_c, ones_c)
        else:
            acc = _sc_msg(p, srcm, dstm, zeros_f)
        p = _tc_update(p, acc, cnt, wlf, wlb, wrf, wrb,
                       (0.5 * (blf + blb)).reshape(1, F))

    sq = params["squad"]
    roh = jnp.pad(jax.nn.one_hot(role_idx, 5, dtype=f32).reshape(22, 5),
                  ((0, 2), (0, 3)))
    boh = jnp.pad(jax.nn.one_hot(bat_idx, 3, dtype=f32).reshape(22, 3),
                  ((0, 2), (0, 5)))
    woh = jnp.pad(jax.nn.one_hot(bowl_idx, 9, dtype=f32).reshape(22, 9),
                  ((0, 2), (0, 7)))
    expf = jnp.pad(exp_feats.reshape(22, 4), ((0, 2), (0, 4)))
    rt = jnp.pad(sq["role"], ((0, 3), (0, 0)))
    bt = jnp.pad(sq["bat"], ((0, 5), (0, 0)))
    wt = jnp.pad(sq["bowl"], ((0, 7), (0, 0)))
    eW, eb2 = sq["exp"]
    eWp = jnp.pad(eW, ((0, 4), (0, 0)))
    aggW, aggb = sq["agg"]
    pW, pb = params["proj"]
    Ww, bw = params["weather_enc"]
    Wve, bve = params["venue_enc"]
    at = params["attn"]
    (W1, b1), (W2, b2), (W3, b3) = params["mp"]
    (Wa, ba), (Wb2, bb2) = params["wip"]

    mp, wip, pe, att, ve, te = _tc_head((
        p, roh, boh, woh, expf,
        rt, bt, wt, eWp, eb2.reshape(1, 8),
        aggW.reshape(4, 8, 32), aggb.reshape(1, 32), pW, pb.reshape(1, F),
        weather_features.reshape(1, 16), Ww, bw.reshape(1, F),
        venue_features.reshape(1, 8), Wve, bve.reshape(1, F),
        at["Wv"][0], at["Wv"][1].reshape(1, F),
        at["Wo"][0], at["Wo"][1].reshape(1, F),
        W1.reshape(4, F, 2 * F), b1.reshape(1, 2 * F),
        W2, b2.reshape(1, F), W3, b3.reshape(1, 1),
        Wa, ba.reshape(1, 32), Wb2, bb2.reshape(1, 3),
    ))
    return (mp.reshape(1), wip.reshape(3), pe.reshape(F), att.reshape(F),
            ve.reshape(F), te.reshape(F))


# speedup vs baseline: 3.5953x; 1.0093x over previous
import functools

import jax
import jax.numpy as jnp
from jax import lax
from jax.experimental import pallas as pl
from jax.experimental.pallas import tpu as pltpu
from jax.experimental.pallas import tpu_sc as plsc

N_P = 10000
P_ROWS = 10240
NE = 320000
NC, NS = 2, 16
NW = NC * NS
CHUNK = 128
NE_PAD = 327680
CPT = NE_PAD // (NW * CHUNK)
TROWS = P_ROWS // NS
CNT_W = 16
F = 64


CPT2 = 2 * CPT


NBUF = 4
NBUF2 = 5


def _msg_loop(table, src_v, dst_v, rows, gsems, ssems, acc):
    nb = len(gsems)

    def body(j, carry):
        for b in range(nb):
            k = nb * j + b

            @pl.when(j > 0)
            def _():
                pltpu.make_async_copy(
                    rows.at[b], acc.at[dst_v.at[k - nb]], ssems[b]).wait()

            pltpu.async_copy(table.at[src_v.at[k]], rows.at[b], gsems[b])
        for b in range(nb):
            k = nb * j + b
            pltpu.make_async_copy(
                table.at[src_v.at[k]], rows.at[b], gsems[b]).wait()
            pltpu.async_copy(rows.at[b], acc.at[dst_v.at[k]], ssems[b],
                             add=True)
        return carry

    lax.fori_loop(0, CPT2 // nb, body, 0)
    for b in range(nb):
        pltpu.make_async_copy(
            rows.at[b], acc.at[dst_v.at[CPT2 - nb + b]], ssems[b]).wait()


def _sc_msg_body(table, srcm, dstm, zeros, out, src_v, dst_v, rows,
                 g0, g1, g2, g3, g4, s0, s1, s2, s3, s4, acc):
    c = lax.axis_index("c")
    s = lax.axis_index("s")
    pltpu.sync_copy(zeros, acc.at[pl.ds(s * TROWS, TROWS)])
    pltpu.sync_copy(srcm.at[c, s], src_v)
    pltpu.sync_copy(dstm.at[c, s], dst_v)
    plsc.subcore_barrier()
    _msg_loop(table, src_v, dst_v, rows, (g0, g1, g2, g3, g4),
              (s0, s1, s2, s3, s4), acc)
    plsc.subcore_barrier()
    pltpu.sync_copy(acc.at[pl.ds(s * TROWS, TROWS)], out.at[c, s])


def _sc_msg1_body(table, srcm, dstm, zeros, zeros_c, ones, out, cnt_out,
                  src_v, dst_v, rows, ones_v, g0, g1, g2, g3, s0, s1, s2, s3,
                  acc, cnt):
    c = lax.axis_index("c")
    s = lax.axis_index("s")
    pltpu.sync_copy(zeros, acc.at[pl.ds(s * TROWS, TROWS)])
    pltpu.sync_copy(zeros_c, cnt.at[pl.ds(s * CTROWS, CTROWS)])
    pltpu.sync_copy(srcm.at[c, s], src_v)
    pltpu.sync_copy(dstm.at[c, s], dst_v)
    pltpu.sync_copy(ones, ones_v)
    plsc.subcore_barrier()
    _msg_loop(table, src_v, dst_v, rows, (g0, g1, g2, g3), (s0, s1, s2, s3),
              acc)

    def cbody(j, carry):
        pltpu.sync_copy(ones_v, cnt.at[dst_v.at[j]], add=True)
        return carry

    lax.fori_loop(0, CPT2, cbody, 0)
    plsc.subcore_barrier()
    pltpu.sync_copy(acc.at[pl.ds(s * TROWS, TROWS)], out.at[c, s])
    pltpu.sync_copy(cnt.at[pl.ds(s * CTROWS, CTROWS)], cnt_out.at[c, s])


CTROWS = P_ROWS // NS


@functools.lru_cache(maxsize=None)
def _build_sc_kernels():
    mesh = plsc.VectorSubcoreMesh(core_axis_name="c", subcore_axis_name="s",
                                  num_cores=NC, num_subcores=NS)
    idxs = [
        pltpu.VMEM((CPT2, CHUNK), jnp.int32),
        pltpu.VMEM((CPT2, CHUNK), jnp.int32),
    ]
    msg = pl.kernel(
        _sc_msg_body,
        jax.ShapeDtypeStruct((NC, NS, TROWS, F), jnp.float32),
        mesh=mesh,
        compiler_params=pltpu.CompilerParams(use_tc_tiling_on_sc=False),
        scratch_types=idxs + [
            pltpu.VMEM((NBUF2, CHUNK, F), jnp.float32),
        ] + [pltpu.SemaphoreType.DMA] * (2 * NBUF2) + [
            pltpu.VMEM_SHARED((P_ROWS, F), jnp.float32),
        ],
    )
    msg1 = pl.kernel(
        _sc_msg1_body,
        (jax.ShapeDtypeStruct((NC, NS, TROWS, F), jnp.float32),
         jax.ShapeDtypeStruct((NC, NS, CTROWS, CNT_W), jnp.float32)),
        mesh=mesh,
        compiler_params=pltpu.CompilerParams(use_tc_tiling_on_sc=False),
        scratch_types=idxs + [
            pltpu.VMEM((NBUF, CHUNK, F), jnp.float32),
            pltpu.VMEM((CHUNK, CNT_W), jnp.float32),
        ] + [pltpu.SemaphoreType.DMA] * (2 * NBUF) + [
            pltpu.VMEM_SHARED((P_ROWS, F), jnp.float32),
            pltpu.VMEM_SHARED((P_ROWS, CNT_W), jnp.float32),
        ],
    )
    return msg, msg1


_BR = 1024


def _enc_body(x_ref, w_ref, b_ref, o_ref):
    o_ref[...] = (
        jnp.dot(x_ref[...], w_ref[...], preferred_element_type=jnp.float32)
        + b_ref[...]
    )


def _tc_enc(x, w, b):
    nblk = P_ROWS // _BR
    return pl.pallas_call(
        _enc_body,
        grid=(nblk,),
        in_specs=[
            pl.BlockSpec((_BR, 128), lambda i: (i, 0)),
            pl.BlockSpec((128, F), lambda i: (0, 0)),
            pl.BlockSpec((1, F), lambda i: (0, 0)),
        ],
        out_specs=pl.BlockSpec((_BR, F), lambda i: (i, 0)),
        out_shape=jax.ShapeDtypeStruct((P_ROWS, F), jnp.float32),
    )(x, w, b)


def _upd_body(p_ref, af_ref, ab_ref, cf_ref, cb_ref, wlf_ref, wlb_ref,
              wrf_ref, wrb_ref, bl_ref, o_ref):
    aggf = af_ref[0, 0] * (0.5 / jnp.maximum(cf_ref[0, 0, :, 0:1], 1.0))
    aggb = ab_ref[0, 0] * (0.5 / jnp.maximum(cb_ref[0, 0, :, 0:1], 1.0))
    wr = 0.5 * (wrf_ref[...] + wrb_ref[...])
    acc = (
        jnp.dot(aggf, wlf_ref[...], preferred_element_type=jnp.float32)
        + jnp.dot(aggb, wlb_ref[...], preferred_element_type=jnp.float32)
        + jnp.dot(p_ref[...], wr, preferred_element_type=jnp.float32)
        + bl_ref[...]
    )
    o_ref[...] = jnp.maximum(acc, 0.0)


def _tc_update(p, acc, cnt, wlf, wlb, wrf, wrb, bl):
    mat = lambda: pl.BlockSpec((F, F), lambda i: (0, 0))
    return pl.pallas_call(
        _upd_body,
        grid=(NS,),
        in_specs=[
            pl.BlockSpec((TROWS, F), lambda i: (i, 0)),
            pl.BlockSpec((1, 1, TROWS, F), lambda i: (0, i, 0, 0)),
            pl.BlockSpec((1, 1, TROWS, F), lambda i: (1, i, 0, 0)),
            pl.BlockSpec((1, 1, CTROWS, CNT_W), lambda i: (0, i, 0, 0)),
            pl.BlockSpec((1, 1, CTROWS, CNT_W), lambda i: (1, i, 0, 0)),
            mat(), mat(), mat(), mat(),
            pl.BlockSpec((1, F), lambda i: (0, 0)),
        ],
        out_specs=pl.BlockSpec((TROWS, F), lambda i: (i, 0)),
        out_shape=jax.ShapeDtypeStruct((P_ROWS, F), jnp.float32),
    )(p, acc, acc, cnt, cnt, wlf, wlb, wrf, wrb, bl)


def _head_body(p3_ref, roh_ref, boh_ref, woh_ref, exp_ref,
               rt_ref, bt_ref, wt_ref, ew_ref, eb_ref,
               aggw_ref, aggb_ref, pw_ref, pb_ref,
               wf_ref, ww_ref, bw_ref, vf_ref, wv_ref, bv_ref,
               awv_ref, abv_ref, awo_ref, abo_ref,
               w1_ref, b1_ref, w2_ref, b2_ref, w3_ref, b3_ref,
               wa_ref, ba_ref, wb_ref, bb_ref,
               mp_ref, wip_ref, pe_ref, att_ref, ve_ref, te_ref):
    f32 = jnp.float32
    dot = functools.partial(jnp.dot, preferred_element_type=f32)
    rows = lax.broadcasted_iota(jnp.int32, (P_ROWS, 1), 0)
    pe = jnp.sum(jnp.where(rows < N_P, p3_ref[...], 0.0), axis=0,
                 keepdims=True) * (1.0 / N_P)
    role_e = dot(roh_ref[...], rt_ref[...])
    bat_e = dot(boh_ref[...], bt_ref[...])
    bowl_e = dot(woh_ref[...], wt_ref[...])
    exp_e = dot(exp_ref[...], ew_ref[...]) + eb_ref[...]
    t_i = lax.broadcasted_iota(jnp.int32, (2, 24), 0)
    r_i = lax.broadcasted_iota(jnp.int32, (2, 24), 1)
    sel = jnp.where((r_i >= 11 * t_i) & (r_i < 11 * t_i + 11),
                    f32(1.0 / 11.0), f32(0.0))
    sq = (
        dot(dot(sel, role_e), aggw_ref[0])
        + dot(dot(sel, bat_e), aggw_ref[1])
        + dot(dot(sel, bowl_e), aggw_ref[2])
        + dot(dot(sel, exp_e), aggw_ref[3])
        + aggb_ref[...]
    )
    te = dot(0.5 * jnp.sum(sq, axis=0, keepdims=True), pw_ref[...]) + pb_ref[...]
    wemb = dot(wf_ref[...], ww_ref[...]) + bw_ref[...]
    att = dot(dot(wemb, awv_ref[...]) + abv_ref[...], awo_ref[...]) + abo_ref[...]
    ve = dot(vf_ref[...], wv_ref[...]) + bv_ref[...]
    h = (
        dot(pe, w1_ref[0]) + dot(att, w1_ref[1]) + dot(ve, w1_ref[2])
        + dot(te, w1_ref[3]) + b1_ref[...]
    )
    h = jnp.maximum(h, 0.0)
    h = jnp.maximum(dot(h, w2_ref[...]) + b2_ref[...], 0.0)
    mp_ref[...] = dot(h, w3_ref[...]) + b3_ref[...]
    wip_ref[...] = (
        dot(jnp.maximum(dot(att, wa_ref[...]) + ba_ref[...], 0.0), wb_ref[...])
        + bb_ref[...]
    )
    pe_ref[...] = pe
    att_ref[...] = att
    ve_ref[...] = ve
    te_ref[...] = te


def _tc_head(args):
    outs = (
        jax.ShapeDtypeStruct((1, 1), jnp.float32),
        jax.ShapeDtypeStruct((1, 3), jnp.float32),
        jax.ShapeDtypeStruct((1, F), jnp.float32),
        jax.ShapeDtypeStruct((1, F), jnp.float32),
        jax.ShapeDtypeStruct((1, F), jnp.float32),
        jax.ShapeDtypeStruct((1, F), jnp.float32),
    )
    return pl.pallas_call(_head_body, out_shape=outs)(*args)


def _prep_idx(src, dst):
    pad = NE_PAD - NE
    spread = jnp.arange(pad, dtype=jnp.int32)
    src = jnp.concatenate([src.astype(jnp.int32), spread % N_P])
    dst = jnp.concatenate(
        [dst.astype(jnp.int32), N_P + spread % (P_ROWS - N_P)])
    return src.reshape(NS, CPT2, CHUNK), dst.reshape(NS, CPT2, CHUNK)


def kernel(x_player, x_venue, x_team, x_match, x_weather, ei_faced,
           ei_bowled_to, ei_played_at_pv, ei_plays_for, ei_played_at_mv,
           ei_had_weather, ei_played_in, weather_features, venue_features,
           role_idx, bat_idx, bowl_idx, exp_feats, params):
    f32 = jnp.float32
    _sc_msg, _sc_msg1 = _build_sc_kernels()
    sf, df = _prep_idx(ei_faced[0], ei_faced[1])
    sb, db = _prep_idx(ei_bowled_to[0], ei_bowled_to[1])
    srcm = jnp.stack([sf, sb])
    dstm = jnp.stack([df, db])

    zeros_f = jnp.zeros((TROWS, F), f32)
    zeros_c = jnp.zeros((CTROWS, CNT_W), f32)
    ones_c = jnp.ones((CHUNK, CNT_W), f32)

    ew, ebias = params["enc"]["player"]
    xp = jnp.pad(x_player, ((0, P_ROWS - N_P), (0, 0)))
    p = _tc_enc(xp, ew, ebias.reshape(1, F))

    cnt = None
    for li, ((wlf, blf, wrf), (wlb, blb, wrb)) in enumerate(
            (layer[0], layer[1]) for layer in params["convs"]):
        if li == 0:
            acc, cnt = _sc_msg1(p, srcm, dstm, zeros_f, zeros_c, ones_c)
        else:
            acc = _sc_msg(p, srcm, dstm, zeros_f)
        p = _tc_update(p, acc, cnt, wlf, wlb, wrf, wrb,
                       (0.5 * (blf + blb)).reshape(1, F))

    sq = params["squad"]
    roh = jnp.pad(jax.nn.one_hot(role_idx, 5, dtype=f32).reshape(22, 5),
                  ((0, 2), (0, 3)))
    boh = jnp.pad(jax.nn.one_hot(bat_idx, 3, dtype=f32).reshape(22, 3),
                  ((0, 2), (0, 5)))
    woh = jnp.pad(jax.nn.one_hot(bowl_idx, 9, dtype=f32).reshape(22, 9),
                  ((0, 2), (0, 7)))
    expf = jnp.pad(exp_feats.reshape(22, 4), ((0, 2), (0, 4)))
    rt = jnp.pad(sq["role"], ((0, 3), (0, 0)))
    bt = jnp.pad(sq["bat"], ((0, 5), (0, 0)))
    wt = jnp.pad(sq["bowl"], ((0, 7), (0, 0)))
    eW, eb2 = sq["exp"]
    eWp = jnp.pad(eW, ((0, 4), (0, 0)))
    aggW, aggb = sq["agg"]
    pW, pb = params["proj"]
    Ww, bw = params["weather_enc"]
    Wve, bve = params["venue_enc"]
    at = params["attn"]
    (W1, b1), (W2, b2), (W3, b3) = params["mp"]
    (Wa, ba), (Wb2, bb2) = params["wip"]

    mp, wip, pe, att, ve, te = _tc_head((
        p, roh, boh, woh, expf,
        rt, bt, wt, eWp, eb2.reshape(1, 8),
        aggW.reshape(4, 8, 32), aggb.reshape(1, 32), pW, pb.reshape(1, F),
        weather_features.reshape(1, 16), Ww, bw.reshape(1, F),
        venue_features.reshape(1, 8), Wve, bve.reshape(1, F),
        at["Wv"][0], at["Wv"][1].reshape(1, F),
        at["Wo"][0], at["Wo"][1].reshape(1, F),
        W1.reshape(4, F, 2 * F), b1.reshape(1, 2 * F),
        W2, b2.reshape(1, F), W3, b3.reshape(1, 1),
        Wa, ba.reshape(1, 32), Wb2, bb2.reshape(1, 3),
    ))
    return (mp.reshape(1), wip.reshape(3), pe.reshape(F), att.reshape(F),
            ve.reshape(F), te.reshape(F))


# speedup vs baseline: 3.6362x; 1.0114x over previous
import functools

import jax
import jax.numpy as jnp
from jax import lax
from jax.experimental import pallas as pl
from jax.experimental.pallas import tpu as pltpu
from jax.experimental.pallas import tpu_sc as plsc

N_P = 10000
P_ROWS = 10240
NE = 320000
NC, NS = 2, 16
NW = NC * NS
CHUNK = 128
NE_PAD = 327680
CPT = NE_PAD // (NW * CHUNK)
TROWS = P_ROWS // NS
CNT_W = 8
F = 64


CPT2 = 2 * CPT


NBUF = 4
NBUF2 = 5


def _msg_loop(table, src_v, dst_v, rows, gsems, ssems, acc):
    nb = len(gsems)

    def body(j, carry):
        for b in range(nb):
            k = nb * j + b

            @pl.when(j > 0)
            def _():
                pltpu.make_async_copy(
                    rows.at[b], acc.at[dst_v.at[k - nb]], ssems[b]).wait()

            pltpu.async_copy(table.at[src_v.at[k]], rows.at[b], gsems[b])
        for b in range(nb):
            k = nb * j + b
            pltpu.make_async_copy(
                table.at[src_v.at[k]], rows.at[b], gsems[b]).wait()
            pltpu.async_copy(rows.at[b], acc.at[dst_v.at[k]], ssems[b],
                             add=True)
        return carry

    lax.fori_loop(0, CPT2 // nb, body, 0)
    for b in range(nb):
        pltpu.make_async_copy(
            rows.at[b], acc.at[dst_v.at[CPT2 - nb + b]], ssems[b]).wait()


def _sc_msg_body(table, srcm, dstm, zeros, out, src_v, dst_v, rows,
                 g0, g1, g2, g3, g4, s0, s1, s2, s3, s4, acc):
    c = lax.axis_index("c")
    s = lax.axis_index("s")
    pltpu.sync_copy(zeros, acc.at[pl.ds(s * TROWS, TROWS)])
    pltpu.sync_copy(srcm.at[c, s], src_v)
    pltpu.sync_copy(dstm.at[c, s], dst_v)
    plsc.subcore_barrier()
    _msg_loop(table, src_v, dst_v, rows, (g0, g1, g2, g3, g4),
              (s0, s1, s2, s3, s4), acc)
    plsc.subcore_barrier()
    pltpu.sync_copy(acc.at[pl.ds(s * TROWS, TROWS)], out.at[c, s])


def _sc_msg1_body(table, srcm, dstm, zeros, zeros_c, ones, out, cnt_out,
                  src_v, dst_v, rows, ones_v, g0, g1, g2, g3, s0, s1, s2, s3,
                  acc, cnt):
    c = lax.axis_index("c")
    s = lax.axis_index("s")
    pltpu.sync_copy(zeros, acc.at[pl.ds(s * TROWS, TROWS)])
    pltpu.sync_copy(zeros_c, cnt.at[pl.ds(s * CTROWS, CTROWS)])
    pltpu.sync_copy(srcm.at[c, s], src_v)
    pltpu.sync_copy(dstm.at[c, s], dst_v)
    pltpu.sync_copy(ones, ones_v)
    plsc.subcore_barrier()
    _msg_loop(table, src_v, dst_v, rows, (g0, g1, g2, g3), (s0, s1, s2, s3),
              acc)

    def cbody(j, carry):
        pltpu.sync_copy(ones_v, cnt.at[dst_v.at[j]], add=True)
        return carry

    lax.fori_loop(0, CPT2, cbody, 0)
    plsc.subcore_barrier()
    pltpu.sync_copy(acc.at[pl.ds(s * TROWS, TROWS)], out.at[c, s])
    pltpu.sync_copy(cnt.at[pl.ds(s * CTROWS, CTROWS)], cnt_out.at[c, s])


CTROWS = P_ROWS // NS


@functools.lru_cache(maxsize=None)
def _build_sc_kernels():
    mesh = plsc.VectorSubcoreMesh(core_axis_name="c", subcore_axis_name="s",
                                  num_cores=NC, num_subcores=NS)
    idxs = [
        pltpu.VMEM((CPT2, CHUNK), jnp.int32),
        pltpu.VMEM((CPT2, CHUNK), jnp.int32),
    ]
    msg = pl.kernel(
        _sc_msg_body,
        jax.ShapeDtypeStruct((NC, NS, TROWS, F), jnp.float32),
        mesh=mesh,
        compiler_params=pltpu.CompilerParams(use_tc_tiling_on_sc=False),
        scratch_types=idxs + [
            pltpu.VMEM((NBUF2, CHUNK, F), jnp.float32),
        ] + [pltpu.SemaphoreType.DMA] * (2 * NBUF2) + [
            pltpu.VMEM_SHARED((P_ROWS, F), jnp.float32),
        ],
    )
    msg1 = pl.kernel(
        _sc_msg1_body,
        (jax.ShapeDtypeStruct((NC, NS, TROWS, F), jnp.float32),
         jax.ShapeDtypeStruct((NC, NS, CTROWS, CNT_W), jnp.float32)),
        mesh=mesh,
        compiler_params=pltpu.CompilerParams(use_tc_tiling_on_sc=False),
        scratch_types=idxs + [
            pltpu.VMEM((NBUF, CHUNK, F), jnp.float32),
            pltpu.VMEM((CHUNK, CNT_W), jnp.float32),
        ] + [pltpu.SemaphoreType.DMA] * (2 * NBUF) + [
            pltpu.VMEM_SHARED((P_ROWS, F), jnp.float32),
            pltpu.VMEM_SHARED((P_ROWS, CNT_W), jnp.float32),
        ],
    )
    return msg, msg1


_BR = 1024


def _enc_body(x_ref, w_ref, b_ref, o_ref):
    o_ref[...] = (
        jnp.dot(x_ref[...], w_ref[...], preferred_element_type=jnp.float32)
        + b_ref[...]
    )


def _tc_enc(x, w, b):
    nblk = P_ROWS // _BR
    return pl.pallas_call(
        _enc_body,
        grid=(nblk,),
        in_specs=[
            pl.BlockSpec((_BR, 128), lambda i: (i, 0)),
            pl.BlockSpec((128, F), lambda i: (0, 0)),
            pl.BlockSpec((1, F), lambda i: (0, 0)),
        ],
        out_specs=pl.BlockSpec((_BR, F), lambda i: (i, 0)),
        out_shape=jax.ShapeDtypeStruct((P_ROWS, F), jnp.float32),
    )(x, w, b)


def _upd_body(p_ref, af_ref, ab_ref, cf_ref, cb_ref, wlf_ref, wlb_ref,
              wrf_ref, wrb_ref, bl_ref, o_ref):
    aggf = af_ref[0, 0] * (0.5 / jnp.maximum(cf_ref[0, 0, :, 0:1], 1.0))
    aggb = ab_ref[0, 0] * (0.5 / jnp.maximum(cb_ref[0, 0, :, 0:1], 1.0))
    wr = 0.5 * (wrf_ref[...] + wrb_ref[...])
    acc = (
        jnp.dot(aggf, wlf_ref[...], preferred_element_type=jnp.float32)
        + jnp.dot(aggb, wlb_ref[...], preferred_element_type=jnp.float32)
        + jnp.dot(p_ref[...], wr, preferred_element_type=jnp.float32)
        + bl_ref[...]
    )
    o_ref[...] = jnp.maximum(acc, 0.0)


def _tc_update(p, acc, cnt, wlf, wlb, wrf, wrb, bl):
    mat = lambda: pl.BlockSpec((F, F), lambda i: (0, 0))
    return pl.pallas_call(
        _upd_body,
        grid=(NS,),
        in_specs=[
            pl.BlockSpec((TROWS, F), lambda i: (i, 0)),
            pl.BlockSpec((1, 1, TROWS, F), lambda i: (0, i, 0, 0)),
            pl.BlockSpec((1, 1, TROWS, F), lambda i: (1, i, 0, 0)),
            pl.BlockSpec((1, 1, CTROWS, CNT_W), lambda i: (0, i, 0, 0)),
            pl.BlockSpec((1, 1, CTROWS, CNT_W), lambda i: (1, i, 0, 0)),
            mat(), mat(), mat(), mat(),
            pl.BlockSpec((1, F), lambda i: (0, 0)),
        ],
        out_specs=pl.BlockSpec((TROWS, F), lambda i: (i, 0)),
        out_shape=jax.ShapeDtypeStruct((P_ROWS, F), jnp.float32),
    )(p, acc, acc, cnt, cnt, wlf, wlb, wrf, wrb, bl)


def _head_body(p3_ref, roh_ref, boh_ref, woh_ref, exp_ref,
               rt_ref, bt_ref, wt_ref, ew_ref, eb_ref,
               aggw_ref, aggb_ref, pw_ref, pb_ref,
               wf_ref, ww_ref, bw_ref, vf_ref, wv_ref, bv_ref,
               awv_ref, abv_ref, awo_ref, abo_ref,
               w1_ref, b1_ref, w2_ref, b2_ref, w3_ref, b3_ref,
               wa_ref, ba_ref, wb_ref, bb_ref,
               mp_ref, wip_ref, pe_ref, att_ref, ve_ref, te_ref):
    f32 = jnp.float32
    dot = functools.partial(jnp.dot, preferred_element_type=f32)
    rows = lax.broadcasted_iota(jnp.int32, (P_ROWS, 1), 0)
    pe = jnp.sum(jnp.where(rows < N_P, p3_ref[...], 0.0), axis=0,
                 keepdims=True) * (1.0 / N_P)
    role_e = dot(roh_ref[...], rt_ref[...])
    bat_e = dot(boh_ref[...], bt_ref[...])
    bowl_e = dot(woh_ref[...], wt_ref[...])
    exp_e = dot(exp_ref[...], ew_ref[...]) + eb_ref[...]
    t_i = lax.broadcasted_iota(jnp.int32, (2, 24), 0)
    r_i = lax.broadcasted_iota(jnp.int32, (2, 24), 1)
    sel = jnp.where((r_i >= 11 * t_i) & (r_i < 11 * t_i + 11),
                    f32(1.0 / 11.0), f32(0.0))
    sq = (
        dot(dot(sel, role_e), aggw_ref[0])
        + dot(dot(sel, bat_e), aggw_ref[1])
        + dot(dot(sel, bowl_e), aggw_ref[2])
        + dot(dot(sel, exp_e), aggw_ref[3])
        + aggb_ref[...]
    )
    te = dot(0.5 * jnp.sum(sq, axis=0, keepdims=True), pw_ref[...]) + pb_ref[...]
    wemb = dot(wf_ref[...], ww_ref[...]) + bw_ref[...]
    att = dot(dot(wemb, awv_ref[...]) + abv_ref[...], awo_ref[...]) + abo_ref[...]
    ve = dot(vf_ref[...], wv_ref[...]) + bv_ref[...]
    h = (
        dot(pe, w1_ref[0]) + dot(att, w1_ref[1]) + dot(ve, w1_ref[2])
        + dot(te, w1_ref[3]) + b1_ref[...]
    )
    h = jnp.maximum(h, 0.0)
    h = jnp.maximum(dot(h, w2_ref[...]) + b2_ref[...], 0.0)
    mp_ref[...] = dot(h, w3_ref[...]) + b3_ref[...]
    wip_ref[...] = (
        dot(jnp.maximum(dot(att, wa_ref[...]) + ba_ref[...], 0.0), wb_ref[...])
        + bb_ref[...]
    )
    pe_ref[...] = pe
    att_ref[...] = att
    ve_ref[...] = ve
    te_ref[...] = te


def _tc_head(args):
    outs = (
        jax.ShapeDtypeStruct((1, 1), jnp.float32),
        jax.ShapeDtypeStruct((1, 3), jnp.float32),
        jax.ShapeDtypeStruct((1, F), jnp.float32),
        jax.ShapeDtypeStruct((1, F), jnp.float32),
        jax.ShapeDtypeStruct((1, F), jnp.float32),
        jax.ShapeDtypeStruct((1, F), jnp.float32),
    )
    return pl.pallas_call(_head_body, out_shape=outs)(*args)


def _prep_idx(src, dst):
    pad = NE_PAD - NE
    spread = jnp.arange(pad, dtype=jnp.int32)
    src = jnp.concatenate([src.astype(jnp.int32), spread % N_P])
    dst = jnp.concatenate(
        [dst.astype(jnp.int32), N_P + spread % (P_ROWS - N_P)])
    return src.reshape(NS, CPT2, CHUNK), dst.reshape(NS, CPT2, CHUNK)


def kernel(x_player, x_venue, x_team, x_match, x_weather, ei_faced,
           ei_bowled_to, ei_played_at_pv, ei_plays_for, ei_played_at_mv,
           ei_had_weather, ei_played_in, weather_features, venue_features,
           role_idx, bat_idx, bowl_idx, exp_feats, params):
    f32 = jnp.float32
    _sc_msg, _sc_msg1 = _build_sc_kernels()
    sf, df = _prep_idx(ei_faced[0], ei_faced[1])
    sb, db = _prep_idx(ei_bowled_to[0], ei_bowled_to[1])
    srcm = jnp.stack([sf, sb])
    dstm = jnp.stack([df, db])

    zeros_f = jnp.zeros((TROWS, F), f32)
    zeros_c = jnp.zeros((CTROWS, CNT_W), f32)
    ones_c = jnp.ones((CHUNK, CNT_W), f32)

    ew, ebias = params["enc"]["player"]
    xp = jnp.pad(x_player, ((0, P_ROWS - N_P), (0, 0)))
    p = _tc_enc(xp, ew, ebias.reshape(1, F))

    cnt = None
    for li, ((wlf, blf, wrf), (wlb, blb, wrb)) in enumerate(
            (layer[0], layer[1]) for layer in params["convs"]):
        if li == 0:
            acc, cnt = _sc_msg1(p, srcm, dstm, zeros_f, zeros_c, ones_c)
        else:
            acc = _sc_msg(p, srcm, dstm, zeros_f)
        p = _tc_update(p, acc, cnt, wlf, wlb, wrf, wrb,
                       (0.5 * (blf + blb)).reshape(1, F))

    sq = params["squad"]
    roh = jnp.pad(jax.nn.one_hot(role_idx, 5, dtype=f32).reshape(22, 5),
                  ((0, 2), (0, 3)))
    boh = jnp.pad(jax.nn.one_hot(bat_idx, 3, dtype=f32).reshape(22, 3),
                  ((0, 2), (0, 5)))
    woh = jnp.pad(jax.nn.one_hot(bowl_idx, 9, dtype=f32).reshape(22, 9),
                  ((0, 2), (0, 7)))
    expf = jnp.pad(exp_feats.reshape(22, 4), ((0, 2), (0, 4)))
    rt = jnp.pad(sq["role"], ((0, 3), (0, 0)))
    bt = jnp.pad(sq["bat"], ((0, 5), (0, 0)))
    wt = jnp.pad(sq["bowl"], ((0, 7), (0, 0)))
    eW, eb2 = sq["exp"]
    eWp = jnp.pad(eW, ((0, 4), (0, 0)))
    aggW, aggb = sq["agg"]
    pW, pb = params["proj"]
    Ww, bw = params["weather_enc"]
    Wve, bve = params["venue_enc"]
    at = params["attn"]
    (W1, b1), (W2, b2), (W3, b3) = params["mp"]
    (Wa, ba), (Wb2, bb2) = params["wip"]

    mp, wip, pe, att, ve, te = _tc_head((
        p, roh, boh, woh, expf,
        rt, bt, wt, eWp, eb2.reshape(1, 8),
        aggW.reshape(4, 8, 32), aggb.reshape(1, 32), pW, pb.reshape(1, F),
        weather_features.reshape(1, 16), Ww, bw.reshape(1, F),
        venue_features.reshape(1, 8), Wve, bve.reshape(1, F),
        at["Wv"][0], at["Wv"][1].reshape(1, F),
        at["Wo"][0], at["Wo"][1].reshape(1, F),
        W1.reshape(4, F, 2 * F), b1.reshape(1, 2 * F),
        W2, b2.reshape(1, F), W3, b3.reshape(1, 1),
        Wa, ba.reshape(1, 32), Wb2, bb2.reshape(1, 3),
    ))
    return (mp.reshape(1), wip.reshape(3), pe.reshape(F), att.reshape(F),
            ve.reshape(F), te.reshape(F))
